# Initial kernel scaffold; baseline (speedup 1.0000x reference)
import functools

import jax
import jax.numpy as jnp
import numpy as np
from jax import lax
from jax.experimental import pallas as pl
from jax.experimental.pallas import tpu as pltpu
from jax.experimental.pallas import tpu_sc as plsc

_N = 10000
_E = 320000
_H = 128
_R = 32
_H3 = 3 * _H

_SCALED_SILU = 1.0 / 0.6
_INV_SQRT_3 = 1.0 / np.sqrt(3.0)
_INV_SQRT_H = 1.0 / np.sqrt(float(_H))

_NC = 2
_NS = 16
_NW = _NC * _NS
_PERW = _E // _NW
_C = 80
_CHUNKS = _PERW // _C
_RPT = _N // _NS


def _dense_body(x_ref, w1_ref, b1_ref, w2_ref, b2_ref, g_ref, bb_ref, out_ref):
    x = x_ref[...]
    mu = jnp.mean(x, axis=-1, keepdims=True)
    var = jnp.mean((x - mu) ** 2, axis=-1, keepdims=True)
    xln = (x - mu) * lax.rsqrt(var + 1e-5) * g_ref[...] + bb_ref[...]
    h = lax.dot_general(xln, w1_ref[...], (((1,), (1,)), ((), ())),
                        preferred_element_type=jnp.float32) + b1_ref[...]
    h = h * jax.nn.sigmoid(h) * _SCALED_SILU
    xh = lax.dot_general(h, w2_ref[...], (((1,), (1,)), ((), ())),
                         preferred_element_type=jnp.float32) + b2_ref[...]
    scale = jnp.concatenate([
        jnp.ones((1, _H), jnp.float32),
        jnp.full((1, _H), _INV_SQRT_3 * _INV_SQRT_H, jnp.float32),
        jnp.full((1, _H), _INV_SQRT_H, jnp.float32),
    ], axis=1)
    out_ref[...] = xh * scale


def _dense(x, w1, b1, w2, b2, g, bb):
    bn = 2000
    return pl.pallas_call(
        _dense_body,
        grid=(_N // bn,),
        in_specs=[
            pl.BlockSpec((bn, _H), lambda ii: (ii, 0)),
            pl.BlockSpec((_H, _H), lambda ii: (0, 0)),
            pl.BlockSpec((1, _H), lambda ii: (0, 0)),
            pl.BlockSpec((_H3, _H), lambda ii: (0, 0)),
            pl.BlockSpec((1, _H3), lambda ii: (0, 0)),
            pl.BlockSpec((1, _H), lambda ii: (0, 0)),
            pl.BlockSpec((1, _H), lambda ii: (0, 0)),
        ],
        out_specs=pl.BlockSpec((bn, _H3), lambda ii: (ii, 0)),
        out_shape=jax.ShapeDtypeStruct((_N, _H3), jnp.float32),
    )(x, w1, b1, w2, b2, g, bb)


def _rbfh_body(rbf_ref, wr_ref, br_ref, out_ref):
    out_ref[...] = lax.dot_general(
        rbf_ref[...], wr_ref[...], (((1,), (1,)), ((), ())),
        preferred_element_type=jnp.float32) + br_ref[...]


def _rbfh(rbf, wr, br):
    be = 3200
    return pl.pallas_call(
        _rbfh_body,
        grid=(_E // be,),
        in_specs=[
            pl.BlockSpec((be, _R), lambda ii: (ii, 0)),
            pl.BlockSpec((_H3, _R), lambda ii: (0, 0)),
            pl.BlockSpec((1, _H3), lambda ii: (0, 0)),
        ],
        out_specs=pl.BlockSpec((be, _H3), lambda ii: (ii, 0)),
        out_shape=jax.ShapeDtypeStruct((_E, _H3), jnp.float32),
    )(rbf, wr, br)


_MESH = plsc.VectorSubcoreMesh(core_axis_name="c", subcore_axis_name="s")


@functools.partial(
    pl.kernel,
    out_type=(jax.ShapeDtypeStruct((_E, _H3), jnp.float32),
              jax.ShapeDtypeStruct((_E, _H3), jnp.float32)),
    mesh=_MESH,
    scratch_types=[
        pltpu.VMEM((_C,), jnp.int32),
        pltpu.VMEM((_C, _H3), jnp.float32),
        pltpu.VMEM((_C, _H3), jnp.float32),
        pltpu.SemaphoreType.DMA,
        pltpu.SemaphoreType.DMA,
    ],
)
def _sc_gather(xhs, vtab, jidx, xhj_out, vecj_out, jbuf, xbuf, vbuf, sem1, sem2):
    c = lax.axis_index("c")
    s = lax.axis_index("s")
    w = s * _NC + c

    def body(k, carry):
        base = w * _PERW + k * _C
        pltpu.sync_copy(jidx.at[pl.ds(base, _C)], jbuf)
        cp1 = pltpu.async_copy(xhs.at[jbuf], xbuf, sem1)
        cp2 = pltpu.async_copy(vtab.at[jbuf], vbuf, sem2)
        cp1.wait()
        cp2.wait()
        pltpu.sync_copy(xbuf, xhj_out.at[pl.ds(base, _C)])
        pltpu.sync_copy(vbuf, vecj_out.at[pl.ds(base, _C)])
        return carry

    lax.fori_loop(0, _CHUNKS, body, 0)


def _msg_body(xhj_ref, vecj_ref, rbfh_ref, ev_ref, u1_ref, uv0_ref, uv1_ref, uv2_ref):
    m = xhj_ref[...] * rbfh_ref[...]
    m2 = m[:, _H:2 * _H]
    m3 = m[:, 2 * _H:]
    u1_ref[...] = m[:, :_H]
    vecj = vecj_ref[...]
    ev = ev_ref[...]
    for d, ref in enumerate((uv0_ref, uv1_ref, uv2_ref)):
        ref[...] = vecj[:, d * _H:(d + 1) * _H] * m2 + m3 * ev[:, d:d + 1]


def _msg(xhj, vecj, rbfh, ev):
    be = 1600
    out_sds = jax.ShapeDtypeStruct((_E, _H), jnp.float32)
    return pl.pallas_call(
        _msg_body,
        grid=(_E // be,),
        in_specs=[
            pl.BlockSpec((be, _H3), lambda ii: (ii, 0)),
            pl.BlockSpec((be, _H3), lambda ii: (ii, 0)),
            pl.BlockSpec((be, _H3), lambda ii: (ii, 0)),
            pl.BlockSpec((be, 3), lambda ii: (ii, 0)),
        ],
        out_specs=[pl.BlockSpec((be, _H), lambda ii: (ii, 0))] * 4,
        out_shape=[out_sds] * 4,
    )(xhj, vecj, rbfh, ev)


@functools.partial(
    pl.kernel,
    out_type=jax.ShapeDtypeStruct((4, _NC, _N, _H), jnp.float32),
    mesh=_MESH,
    scratch_types=[
        pltpu.VMEM((_C,), jnp.int32),
        pltpu.VMEM((_C, _H), jnp.float32),
        pltpu.VMEM_SHARED((_N, _H), jnp.float32),
        pltpu.SemaphoreType.DMA,
    ],
)
def _sc_scatter(u1, uv0, uv1, uv2, iidx, zeros, out, ibuf, ubuf, acc, sem):
    c = lax.axis_index("c")
    s = lax.axis_index("s")
    w = s * _NC + c
    rbase = s * _RPT

    for p, u in enumerate((u1, uv0, uv1, uv2)):
        pltpu.sync_copy(zeros.at[pl.ds(rbase, _RPT)], acc.at[pl.ds(rbase, _RPT)])
        plsc.subcore_barrier()

        def body(k, carry, u=u):
            base = w * _PERW + k * _C
            pltpu.sync_copy(iidx.at[pl.ds(base, _C)], ibuf)
            pltpu.async_copy(u.at[pl.ds(base, _C)], ubuf, sem).wait()
            pltpu.sync_copy(ubuf, acc.at[ibuf], add=True)
            return carry

        lax.fori_loop(0, _CHUNKS, body, 0)
        plsc.subcore_barrier()
        pltpu.sync_copy(acc.at[pl.ds(rbase, _RPT)],
                        out.at[p, c].at[pl.ds(rbase, _RPT)])
        plsc.subcore_barrier()


def kernel(x, vec, edge_index, edge_rbf, edge_vector, W1, b1, W2, b2, Wr, br,
           ln_g, ln_b):
    xhs = _dense(x, W1, b1.reshape(1, -1), W2, b2.reshape(1, -1),
                 ln_g.reshape(1, -1), ln_b.reshape(1, -1))
    rbfh = _rbfh(edge_rbf, Wr, br.reshape(1, -1))
    j = edge_index[0]
    i = edge_index[1]
    vtab = vec.reshape(_N, _H3)
    xhj, vecj = _sc_gather(xhs, vtab, j)
    u1, uv0, uv1, uv2 = _msg(xhj, vecj, rbfh, edge_vector)
    zeros = jnp.zeros((_N, _H), jnp.float32)
    parts = _sc_scatter(u1, uv0, uv1, uv2, i, zeros)
    dx = parts[0, 0] + parts[0, 1]
    dvec = jnp.stack([parts[1, 0] + parts[1, 1],
                      parts[2, 0] + parts[2, 1],
                      parts[3, 0] + parts[3, 1]], axis=1)
    return dx, dvec


# speedup vs baseline: 13.6784x; 13.6784x over previous
import functools

import jax
import jax.numpy as jnp
import numpy as np
from jax import lax
from jax.experimental import pallas as pl
from jax.experimental.pallas import tpu as pltpu
from jax.experimental.pallas import tpu_sc as plsc

_N = 10000
_E = 320000
_H = 128
_R = 32
_H3 = 3 * _H

_SCALED_SILU = 1.0 / 0.6
_INV_SQRT_3 = 1.0 / np.sqrt(3.0)
_INV_SQRT_H = 1.0 / np.sqrt(float(_H))

_NC = 2
_NS = 16
_NW = _NC * _NS
_PERW = _E // _NW
_C = 80
_CHUNKS = _PERW // _C
_RPT = 624
_RTAIL = _N - _NS * _RPT


def _dense_body(x_ref, w1_ref, b1_ref, w2_ref, b2_ref, g_ref, bb_ref, out_ref):
    x = x_ref[...]
    mu = jnp.mean(x, axis=-1, keepdims=True)
    var = jnp.mean((x - mu) ** 2, axis=-1, keepdims=True)
    xln = (x - mu) * lax.rsqrt(var + 1e-5) * g_ref[...] + bb_ref[...]
    h = lax.dot_general(xln, w1_ref[...], (((1,), (1,)), ((), ())),
                        preferred_element_type=jnp.float32) + b1_ref[...]
    h = h * jax.nn.sigmoid(h) * _SCALED_SILU
    xh = lax.dot_general(h, w2_ref[...], (((1,), (1,)), ((), ())),
                         preferred_element_type=jnp.float32) + b2_ref[...]
    scale = jnp.concatenate([
        jnp.ones((1, _H), jnp.float32),
        jnp.full((1, _H), _INV_SQRT_3 * _INV_SQRT_H, jnp.float32),
        jnp.full((1, _H), _INV_SQRT_H, jnp.float32),
    ], axis=1)
    out_ref[...] = xh * scale


def _dense(x, w1, b1, w2, b2, g, bb):
    bn = 2000
    return pl.pallas_call(
        _dense_body,
        grid=(_N // bn,),
        in_specs=[
            pl.BlockSpec((bn, _H), lambda ii: (ii, 0)),
            pl.BlockSpec((_H, _H), lambda ii: (0, 0)),
            pl.BlockSpec((1, _H), lambda ii: (0, 0)),
            pl.BlockSpec((_H3, _H), lambda ii: (0, 0)),
            pl.BlockSpec((1, _H3), lambda ii: (0, 0)),
            pl.BlockSpec((1, _H), lambda ii: (0, 0)),
            pl.BlockSpec((1, _H), lambda ii: (0, 0)),
        ],
        out_specs=pl.BlockSpec((bn, _H3), lambda ii: (ii, 0)),
        out_shape=jax.ShapeDtypeStruct((_N, _H3), jnp.float32),
    )(x, w1, b1, w2, b2, g, bb)


def _rbfh_body(rbf_ref, wr_ref, br_ref, out_ref):
    out_ref[...] = lax.dot_general(
        rbf_ref[...], wr_ref[...], (((1,), (1,)), ((), ())),
        preferred_element_type=jnp.float32) + br_ref[...]


def _rbfh(rbf, wr, br):
    be = 3200
    return pl.pallas_call(
        _rbfh_body,
        grid=(_E // be,),
        in_specs=[
            pl.BlockSpec((be, _R), lambda ii: (ii, 0)),
            pl.BlockSpec((_H3, _R), lambda ii: (0, 0)),
            pl.BlockSpec((1, _H3), lambda ii: (0, 0)),
        ],
        out_specs=pl.BlockSpec((be, _H3), lambda ii: (ii, 0)),
        out_shape=jax.ShapeDtypeStruct((_E, _H3), jnp.float32),
    )(rbf, wr, br)


@functools.cache
def _sc_gather_fn():
    mesh = plsc.VectorSubcoreMesh(core_axis_name="c", subcore_axis_name="s",
                                  num_cores=_NC, num_subcores=_NS)

    @functools.partial(
        pl.kernel,
        out_type=(jax.ShapeDtypeStruct((_E, _H3), jnp.float32),
                  jax.ShapeDtypeStruct((_E, _H3), jnp.float32)),
        mesh=mesh,
        scratch_types=[
            pltpu.VMEM((_C,), jnp.int32),
            pltpu.VMEM((_C, _H3), jnp.float32),
            pltpu.VMEM((_C, _H3), jnp.float32),
            pltpu.SemaphoreType.DMA,
            pltpu.SemaphoreType.DMA,
        ],
    )
    def _sc_gather(xhs, vtab, jidx, xhj_out, vecj_out, jbuf, xbuf, vbuf, sem1, sem2):
        c = lax.axis_index("c")
        s = lax.axis_index("s")
        w = s * _NC + c

        def body(k, carry):
            base = w * _PERW + k * _C
            pltpu.sync_copy(jidx.at[pl.ds(base, _C)], jbuf)
            cp1 = pltpu.async_copy(xhs.at[jbuf], xbuf, sem1)
            cp2 = pltpu.async_copy(vtab.at[jbuf], vbuf, sem2)
            cp1.wait()
            cp2.wait()
            pltpu.sync_copy(xbuf, xhj_out.at[pl.ds(base, _C)])
            pltpu.sync_copy(vbuf, vecj_out.at[pl.ds(base, _C)])
            return carry

        lax.fori_loop(0, _CHUNKS, body, 0)

    return _sc_gather


def _msg_body(xhj_ref, vecj_ref, rbfh_ref, ev_ref, u1_ref, uv0_ref, uv1_ref, uv2_ref):
    m = xhj_ref[...] * rbfh_ref[...]
    m2 = m[:, _H:2 * _H]
    m3 = m[:, 2 * _H:]
    u1_ref[...] = m[:, :_H]
    vecj = vecj_ref[...]
    ev = ev_ref[...]
    for d, ref in enumerate((uv0_ref, uv1_ref, uv2_ref)):
        ref[...] = vecj[:, d * _H:(d + 1) * _H] * m2 + m3 * ev[:, d:d + 1]


def _msg(xhj, vecj, rbfh, ev):
    be = 1600
    out_sds = jax.ShapeDtypeStruct((_E, _H), jnp.float32)
    return pl.pallas_call(
        _msg_body,
        grid=(_E // be,),
        in_specs=[
            pl.BlockSpec((be, _H3), lambda ii: (ii, 0)),
            pl.BlockSpec((be, _H3), lambda ii: (ii, 0)),
            pl.BlockSpec((be, _H3), lambda ii: (ii, 0)),
            pl.BlockSpec((be, 3), lambda ii: (ii, 0)),
        ],
        out_specs=[pl.BlockSpec((be, _H), lambda ii: (ii, 0))] * 4,
        out_shape=[out_sds] * 4,
    )(xhj, vecj, rbfh, ev)


@functools.cache
def _sc_scatter_fn():
    mesh = plsc.VectorSubcoreMesh(core_axis_name="c", subcore_axis_name="s",
                                  num_cores=_NC, num_subcores=_NS)

    @functools.partial(
        pl.kernel,
        out_type=jax.ShapeDtypeStruct((4, _NC, _N, _H), jnp.float32),
        mesh=mesh,
        scratch_types=[
            pltpu.VMEM((_C,), jnp.int32),
            pltpu.VMEM((_C, _H), jnp.float32),
            pltpu.VMEM_SHARED((_N, _H), jnp.float32),
            pltpu.SemaphoreType.DMA,
        ],
    )
    def _sc_scatter(u1, uv0, uv1, uv2, iidx, zeros, out, ibuf, ubuf, acc, sem):
        c = lax.axis_index("c")
        s = lax.axis_index("s")
        w = s * _NC + c
        rbase = s * _RPT

        tail = _NS * _RPT

        for p, u in enumerate((u1, uv0, uv1, uv2)):
            pltpu.sync_copy(zeros.at[pl.ds(rbase, _RPT)],
                            acc.at[pl.ds(rbase, _RPT)])

            @pl.when(s == _NS - 1)
            def _zero_tail():
                pltpu.sync_copy(zeros.at[pl.ds(tail, _RTAIL)],
                                acc.at[pl.ds(tail, _RTAIL)])

            plsc.subcore_barrier()

            def body(k, carry, u=u):
                base = w * _PERW + k * _C
                pltpu.sync_copy(iidx.at[pl.ds(base, _C)], ibuf)
                pltpu.async_copy(u.at[pl.ds(base, _C)], ubuf, sem).wait()
                pltpu.sync_copy(ubuf, acc.at[ibuf], add=True)
                return carry

            lax.fori_loop(0, _CHUNKS, body, 0)
            plsc.subcore_barrier()
            pltpu.sync_copy(acc.at[pl.ds(rbase, _RPT)],
                            out.at[p, c].at[pl.ds(rbase, _RPT)])

            @pl.when(s == _NS - 1)
            def _dump_tail():
                pltpu.sync_copy(acc.at[pl.ds(tail, _RTAIL)],
                                out.at[p, c].at[pl.ds(tail, _RTAIL)])

            plsc.subcore_barrier()

    return _sc_scatter


def kernel(x, vec, edge_index, edge_rbf, edge_vector, W1, b1, W2, b2, Wr, br,
           ln_g, ln_b):
    xhs = _dense(x, W1, b1.reshape(1, -1), W2, b2.reshape(1, -1),
                 ln_g.reshape(1, -1), ln_b.reshape(1, -1))
    rbfh = _rbfh(edge_rbf, Wr, br.reshape(1, -1))
    j = edge_index[0]
    i = edge_index[1]
    vtab = vec.reshape(_N, _H3)
    xhj, vecj = _sc_gather_fn()(xhs, vtab, j)
    u1, uv0, uv1, uv2 = _msg(xhj, vecj, rbfh, edge_vector)
    zeros = jnp.zeros((_N, _H), jnp.float32)
    parts = _sc_scatter_fn()(u1, uv0, uv1, uv2, i, zeros)
    dx = parts[0, 0] + parts[0, 1]
    dvec = jnp.stack([parts[1, 0] + parts[1, 1],
                      parts[2, 0] + parts[2, 1],
                      parts[3, 0] + parts[3, 1]], axis=1)
    return dx, dvec


# speedup vs baseline: 15.9855x; 1.1687x over previous
import functools

import jax
import jax.numpy as jnp
import numpy as np
from jax import lax
from jax.experimental import pallas as pl
from jax.experimental.pallas import tpu as pltpu
from jax.experimental.pallas import tpu_sc as plsc

_N = 10000
_E = 320000
_H = 128
_R = 32
_H3 = 3 * _H

_SCALED_SILU = 1.0 / 0.6
_INV_SQRT_3 = 1.0 / np.sqrt(3.0)
_INV_SQRT_H = 1.0 / np.sqrt(float(_H))

_NC = 2
_NS = 16
_NW = _NC * _NS
_PERW = _E // _NW
_C = 40
_CHUNKS = _PERW // _C
_M2 = _CHUNKS // 2
_RPT = 624
_RTAIL = _N - _NS * _RPT


def _dense_body(x_ref, w1_ref, b1_ref, w2_ref, b2_ref, g_ref, bb_ref, out_ref):
    x = x_ref[...]
    mu = jnp.mean(x, axis=-1, keepdims=True)
    var = jnp.mean((x - mu) ** 2, axis=-1, keepdims=True)
    xln = (x - mu) * lax.rsqrt(var + 1e-5) * g_ref[...] + bb_ref[...]
    h = lax.dot_general(xln, w1_ref[...], (((1,), (1,)), ((), ())),
                        preferred_element_type=jnp.float32) + b1_ref[...]
    h = h * jax.nn.sigmoid(h) * _SCALED_SILU
    xh = lax.dot_general(h, w2_ref[...], (((1,), (1,)), ((), ())),
                         preferred_element_type=jnp.float32) + b2_ref[...]
    scale = jnp.concatenate([
        jnp.ones((1, _H), jnp.float32),
        jnp.full((1, _H), _INV_SQRT_3 * _INV_SQRT_H, jnp.float32),
        jnp.full((1, _H), _INV_SQRT_H, jnp.float32),
    ], axis=1)
    out_ref[...] = xh * scale


def _dense(x, w1, b1, w2, b2, g, bb):
    bn = 2000
    return pl.pallas_call(
        _dense_body,
        grid=(_N // bn,),
        in_specs=[
            pl.BlockSpec((bn, _H), lambda ii: (ii, 0)),
            pl.BlockSpec((_H, _H), lambda ii: (0, 0)),
            pl.BlockSpec((1, _H), lambda ii: (0, 0)),
            pl.BlockSpec((_H3, _H), lambda ii: (0, 0)),
            pl.BlockSpec((1, _H3), lambda ii: (0, 0)),
            pl.BlockSpec((1, _H), lambda ii: (0, 0)),
            pl.BlockSpec((1, _H), lambda ii: (0, 0)),
        ],
        out_specs=pl.BlockSpec((bn, _H3), lambda ii: (ii, 0)),
        out_shape=jax.ShapeDtypeStruct((_N, _H3), jnp.float32),
    )(x, w1, b1, w2, b2, g, bb)


def _rbfh_body(rbf_ref, wr_ref, br_ref, out_ref):
    out_ref[...] = lax.dot_general(
        rbf_ref[...], wr_ref[...], (((1,), (1,)), ((), ())),
        preferred_element_type=jnp.float32) + br_ref[...]


def _rbfh(rbf, wr, br):
    be = 3200
    return pl.pallas_call(
        _rbfh_body,
        grid=(_E // be,),
        in_specs=[
            pl.BlockSpec((be, _R), lambda ii: (ii, 0)),
            pl.BlockSpec((_H3, _R), lambda ii: (0, 0)),
            pl.BlockSpec((1, _H3), lambda ii: (0, 0)),
        ],
        out_specs=pl.BlockSpec((be, _H3), lambda ii: (ii, 0)),
        out_shape=jax.ShapeDtypeStruct((_E, _H3), jnp.float32),
    )(rbf, wr, br)


@functools.cache
def _sc_gather_fn():
    mesh = plsc.VectorSubcoreMesh(core_axis_name="c", subcore_axis_name="s",
                                  num_cores=_NC, num_subcores=_NS)

    @functools.partial(
        pl.kernel,
        out_type=(jax.ShapeDtypeStruct((_E, _H3), jnp.float32),
                  jax.ShapeDtypeStruct((_E, _H3), jnp.float32)),
        mesh=mesh,
        scratch_types=[
            pltpu.VMEM((_CHUNKS, _C), jnp.int32),
            pltpu.VMEM((_C, _H3), jnp.float32),
            pltpu.VMEM((_C, _H3), jnp.float32),
            pltpu.VMEM((_C, _H3), jnp.float32),
            pltpu.VMEM((_C, _H3), jnp.float32),
            pltpu.SemaphoreType.DMA,
            pltpu.SemaphoreType.DMA,
            pltpu.SemaphoreType.DMA,
            pltpu.SemaphoreType.DMA,
        ],
    )
    def _sc_gather(xhs, vtab, jidx3, xhj_out, vecj_out, jall,
                   xb0, vb0, xb1, vb1, g0, g1, w0, w1):
        c = lax.axis_index("c")
        s = lax.axis_index("s")
        w = s * _NC + c
        pltpu.sync_copy(jidx3.at[w], jall)

        def base(k):
            return w * _PERW + k * _C

        def start(k, xb, vb, gs):
            pltpu.async_copy(xhs.at[jall.at[k]], xb, gs)
            pltpu.async_copy(vtab.at[jall.at[k]], vb, gs)

        def wait_gathers(k, xb, vb, gs):
            pltpu.make_async_copy(xhs.at[jall.at[k]], xb, gs).wait()
            pltpu.make_async_copy(vtab.at[jall.at[k]], vb, gs).wait()

        def fire_writes(k, xb, vb, ws):
            pltpu.async_copy(xb, xhj_out.at[pl.ds(base(k), _C)], ws)
            pltpu.async_copy(vb, vecj_out.at[pl.ds(base(k), _C)], ws)

        def wait_writes(k, xb, vb, ws):
            pltpu.make_async_copy(xb, xhj_out.at[pl.ds(base(k), _C)], ws).wait()
            pltpu.make_async_copy(vb, vecj_out.at[pl.ds(base(k), _C)], ws).wait()

        start(0, xb0, vb0, g0)
        start(1, xb1, vb1, g1)

        def body(k2, carry):
            ka = 2 * k2
            kb = ka + 1
            wait_gathers(ka, xb0, vb0, g0)
            fire_writes(ka, xb0, vb0, w0)
            wait_writes(ka, xb0, vb0, w0)
            start(ka + 2, xb0, vb0, g0)
            wait_gathers(kb, xb1, vb1, g1)
            fire_writes(kb, xb1, vb1, w1)
            wait_writes(kb, xb1, vb1, w1)
            start(kb + 2, xb1, vb1, g1)
            return carry

        lax.fori_loop(0, _M2 - 1, body, 0)
        for k, xb, vb, gs, ws in ((_CHUNKS - 2, xb0, vb0, g0, w0),
                                  (_CHUNKS - 1, xb1, vb1, g1, w1)):
            wait_gathers(k, xb, vb, gs)
            fire_writes(k, xb, vb, ws)
            wait_writes(k, xb, vb, ws)

    return _sc_gather


def _msg_body(xhj_ref, vecj_ref, rbfh_ref, ev_ref, u1_ref, uv0_ref, uv1_ref, uv2_ref):
    m = xhj_ref[...] * rbfh_ref[...]
    m2 = m[:, _H:2 * _H]
    m3 = m[:, 2 * _H:]
    u1_ref[...] = m[:, :_H]
    vecj = vecj_ref[...]
    ev = ev_ref[...]
    for d, ref in enumerate((uv0_ref, uv1_ref, uv2_ref)):
        ref[...] = vecj[:, d * _H:(d + 1) * _H] * m2 + m3 * ev[:, d:d + 1]


def _msg(xhj, vecj, rbfh, ev):
    be = 1600
    out_sds = jax.ShapeDtypeStruct((_E, _H), jnp.float32)
    return pl.pallas_call(
        _msg_body,
        grid=(_E // be,),
        in_specs=[
            pl.BlockSpec((be, _H3), lambda ii: (ii, 0)),
            pl.BlockSpec((be, _H3), lambda ii: (ii, 0)),
            pl.BlockSpec((be, _H3), lambda ii: (ii, 0)),
            pl.BlockSpec((be, 3), lambda ii: (ii, 0)),
        ],
        out_specs=[pl.BlockSpec((be, _H), lambda ii: (ii, 0))] * 4,
        out_shape=[out_sds] * 4,
    )(xhj, vecj, rbfh, ev)


@functools.cache
def _sc_scatter_fn():
    mesh = plsc.VectorSubcoreMesh(core_axis_name="c", subcore_axis_name="s",
                                  num_cores=_NC, num_subcores=_NS)

    @functools.partial(
        pl.kernel,
        out_type=jax.ShapeDtypeStruct((4, _NC, _N, _H), jnp.float32),
        mesh=mesh,
        scratch_types=[
            pltpu.VMEM((_CHUNKS, _C), jnp.int32),
            pltpu.VMEM((_C, _H), jnp.float32),
            pltpu.VMEM((_C, _H), jnp.float32),
            pltpu.VMEM_SHARED((_N, _H), jnp.float32),
            pltpu.SemaphoreType.DMA,
            pltpu.SemaphoreType.DMA,
        ],
    )
    def _sc_scatter(u1, uv0, uv1, uv2, iidx3, zeros, out,
                    iall, ub0, ub1, acc, r0, r1):
        c = lax.axis_index("c")
        s = lax.axis_index("s")
        w = s * _NC + c
        rbase = s * _RPT

        tail = _NS * _RPT
        pltpu.sync_copy(iidx3.at[w], iall)

        def base(k):
            return w * _PERW + k * _C

        for p, u in enumerate((u1, uv0, uv1, uv2)):
            pltpu.sync_copy(zeros.at[pl.ds(rbase, _RPT)],
                            acc.at[pl.ds(rbase, _RPT)])

            @pl.when(s == _NS - 1)
            def _zero_tail():
                pltpu.sync_copy(zeros.at[pl.ds(tail, _RTAIL)],
                                acc.at[pl.ds(tail, _RTAIL)])

            plsc.subcore_barrier()

            def read(k, ub, rs, u=u):
                pltpu.async_copy(u.at[pl.ds(base(k), _C)], ub, rs)

            def wait_read(k, ub, rs, u=u):
                pltpu.make_async_copy(u.at[pl.ds(base(k), _C)], ub, rs).wait()

            def scat(k, ub):
                pltpu.sync_copy(ub, acc.at[iall.at[k]], add=True)

            read(0, ub0, r0)
            read(1, ub1, r1)

            def body(k2, carry):
                ka = 2 * k2
                kb = ka + 1
                wait_read(ka, ub0, r0)
                scat(ka, ub0)
                read(ka + 2, ub0, r0)
                wait_read(kb, ub1, r1)
                scat(kb, ub1)
                read(kb + 2, ub1, r1)
                return carry

            lax.fori_loop(0, _M2 - 1, body, 0)
            for k, ub, rs in ((_CHUNKS - 2, ub0, r0), (_CHUNKS - 1, ub1, r1)):
                wait_read(k, ub, rs)
                scat(k, ub)
            plsc.subcore_barrier()
            pltpu.sync_copy(acc.at[pl.ds(rbase, _RPT)],
                            out.at[p, c].at[pl.ds(rbase, _RPT)])

            @pl.when(s == _NS - 1)
            def _dump_tail():
                pltpu.sync_copy(acc.at[pl.ds(tail, _RTAIL)],
                                out.at[p, c].at[pl.ds(tail, _RTAIL)])

            plsc.subcore_barrier()

    return _sc_scatter


def kernel(x, vec, edge_index, edge_rbf, edge_vector, W1, b1, W2, b2, Wr, br,
           ln_g, ln_b):
    xhs = _dense(x, W1, b1.reshape(1, -1), W2, b2.reshape(1, -1),
                 ln_g.reshape(1, -1), ln_b.reshape(1, -1))
    rbfh = _rbfh(edge_rbf, Wr, br.reshape(1, -1))
    j = edge_index[0].reshape(_NW, _CHUNKS, _C)
    i = edge_index[1].reshape(_NW, _CHUNKS, _C)
    vtab = vec.reshape(_N, _H3)
    xhj, vecj = _sc_gather_fn()(xhs, vtab, j)
    u1, uv0, uv1, uv2 = _msg(xhj, vecj, rbfh, edge_vector)
    zeros = jnp.zeros((_N, _H), jnp.float32)
    parts = _sc_scatter_fn()(u1, uv0, uv1, uv2, i, zeros)
    dx = parts[0, 0] + parts[0, 1]
    dvec = jnp.stack([parts[1, 0] + parts[1, 1],
                      parts[2, 0] + parts[2, 1],
                      parts[3, 0] + parts[3, 1]], axis=1)
    return dx, dvec


# speedup vs baseline: 19.7132x; 1.2332x over previous
import functools

import jax
import jax.numpy as jnp
import numpy as np
from jax import lax
from jax.experimental import pallas as pl
from jax.experimental.pallas import tpu as pltpu
from jax.experimental.pallas import tpu_sc as plsc

_N = 10000
_E = 320000
_H = 128
_R = 32
_H3 = 3 * _H

_SCALED_SILU = 1.0 / 0.6
_INV_SQRT_3 = 1.0 / np.sqrt(3.0)
_INV_SQRT_H = 1.0 / np.sqrt(float(_H))

_NC = 2
_NS = 16
_NW = _NC * _NS
_PERW = _E // _NW
_C = 40
_CHUNKS = _PERW // _C
_M2 = _CHUNKS // 2
_RPT = 624
_RTAIL = _N - _NS * _RPT


def _dense_body(x_ref, vec_ref, w1_ref, b1_ref, w2_ref, b2_ref, g_ref, bb_ref,
                out_ref):
    x = x_ref[...]
    mu = jnp.mean(x, axis=-1, keepdims=True)
    var = jnp.mean((x - mu) ** 2, axis=-1, keepdims=True)
    xln = (x - mu) * lax.rsqrt(var + 1e-5) * g_ref[...] + bb_ref[...]
    h = lax.dot_general(xln, w1_ref[...], (((1,), (1,)), ((), ())),
                        preferred_element_type=jnp.float32) + b1_ref[...]
    h = h * jax.nn.sigmoid(h) * _SCALED_SILU
    xh = lax.dot_general(h, w2_ref[...], (((1,), (1,)), ((), ())),
                         preferred_element_type=jnp.float32) + b2_ref[...]
    scale = jnp.concatenate([
        jnp.ones((1, _H), jnp.float32),
        jnp.full((1, _H), _INV_SQRT_3 * _INV_SQRT_H, jnp.float32),
        jnp.full((1, _H), _INV_SQRT_H, jnp.float32),
    ], axis=1)
    lo = lax.bitcast_convert_type(
        (xh * scale).astype(jnp.bfloat16), jnp.uint16).astype(jnp.uint32)
    hi = lax.bitcast_convert_type(
        vec_ref[...].astype(jnp.bfloat16), jnp.uint16).astype(jnp.uint32)
    out_ref[...] = lax.bitcast_convert_type(lo | (hi << 16), jnp.int32)


def _dense(x, vecf, w1, b1, w2, b2, g, bb):
    bn = 2000
    return pl.pallas_call(
        _dense_body,
        grid=(_N // bn,),
        in_specs=[
            pl.BlockSpec((bn, _H), lambda ii: (ii, 0)),
            pl.BlockSpec((bn, _H3), lambda ii: (ii, 0)),
            pl.BlockSpec((_H, _H), lambda ii: (0, 0)),
            pl.BlockSpec((1, _H), lambda ii: (0, 0)),
            pl.BlockSpec((_H3, _H), lambda ii: (0, 0)),
            pl.BlockSpec((1, _H3), lambda ii: (0, 0)),
            pl.BlockSpec((1, _H), lambda ii: (0, 0)),
            pl.BlockSpec((1, _H), lambda ii: (0, 0)),
        ],
        out_specs=pl.BlockSpec((bn, _H3), lambda ii: (ii, 0)),
        out_shape=jax.ShapeDtypeStruct((_N, _H3), jnp.int32),
    )(x, vecf, w1, b1, w2, b2, g, bb)


def _rbfh_body(rbf_ref, wr_ref, br_ref, out_ref):
    out_ref[...] = lax.dot_general(
        rbf_ref[...], wr_ref[...], (((1,), (1,)), ((), ())),
        preferred_element_type=jnp.float32) + br_ref[...]


def _rbfh(rbf, wr, br):
    be = 3200
    return pl.pallas_call(
        _rbfh_body,
        grid=(_E // be,),
        in_specs=[
            pl.BlockSpec((be, _R), lambda ii: (ii, 0)),
            pl.BlockSpec((_H3, _R), lambda ii: (0, 0)),
            pl.BlockSpec((1, _H3), lambda ii: (0, 0)),
        ],
        out_specs=pl.BlockSpec((be, _H3), lambda ii: (ii, 0)),
        out_shape=jax.ShapeDtypeStruct((_E, _H3), jnp.float32),
    )(rbf, wr, br)


@functools.cache
def _sc_gather_fn():
    mesh = plsc.VectorSubcoreMesh(core_axis_name="c", subcore_axis_name="s",
                                  num_cores=_NC, num_subcores=_NS)

    @functools.partial(
        pl.kernel,
        out_type=jax.ShapeDtypeStruct((_E, _H3), jnp.int32),
        mesh=mesh,
        scratch_types=[
            pltpu.VMEM((_CHUNKS, _C), jnp.int32),
            pltpu.VMEM((_C, _H3), jnp.int32),
            pltpu.VMEM((_C, _H3), jnp.int32),
            pltpu.SemaphoreType.DMA,
            pltpu.SemaphoreType.DMA,
            pltpu.SemaphoreType.DMA,
            pltpu.SemaphoreType.DMA,
        ],
    )
    def _sc_gather(tab, jidx3, rows_out, jall, xb0, xb1, g0, g1, w0, w1):
        c = lax.axis_index("c")
        s = lax.axis_index("s")
        w = s * _NC + c
        pltpu.sync_copy(jidx3.at[w], jall)

        def base(k):
            return w * _PERW + k * _C

        def start(k, xb, gs):
            pltpu.async_copy(tab.at[jall.at[k]], xb, gs)

        def wait_gather(k, xb, gs):
            pltpu.make_async_copy(tab.at[jall.at[k]], xb, gs).wait()

        def fire_write(k, xb, ws):
            pltpu.async_copy(xb, rows_out.at[pl.ds(base(k), _C)], ws)

        def wait_write(k, xb, ws):
            pltpu.make_async_copy(xb, rows_out.at[pl.ds(base(k), _C)], ws).wait()

        start(0, xb0, g0)
        start(1, xb1, g1)

        def body(k2, carry):
            ka = 2 * k2
            kb = ka + 1
            wait_gather(ka, xb0, g0)
            fire_write(ka, xb0, w0)
            wait_write(ka, xb0, w0)
            start(ka + 2, xb0, g0)
            wait_gather(kb, xb1, g1)
            fire_write(kb, xb1, w1)
            wait_write(kb, xb1, w1)
            start(kb + 2, xb1, g1)
            return carry

        lax.fori_loop(0, _M2 - 1, body, 0)
        for k, xb, gs, ws in ((_CHUNKS - 2, xb0, g0, w0),
                              (_CHUNKS - 1, xb1, g1, w1)):
            wait_gather(k, xb, gs)
            fire_write(k, xb, ws)
            wait_write(k, xb, ws)

    return _sc_gather


def _msg_body(rows_ref, rbfh_ref, ev_ref, u1_ref, uv0_ref, uv1_ref, uv2_ref):
    ru = lax.bitcast_convert_type(rows_ref[...], jnp.uint32)
    lo16 = lax.convert_element_type(ru & 0xFFFF, jnp.uint16)
    hi16 = lax.convert_element_type(ru >> 16, jnp.uint16)
    xhj = lax.bitcast_convert_type(lo16, jnp.bfloat16).astype(jnp.float32)
    vecj = lax.bitcast_convert_type(hi16, jnp.bfloat16).astype(jnp.float32)
    m = xhj * rbfh_ref[...]
    m2 = m[:, _H:2 * _H]
    m3 = m[:, 2 * _H:]
    u1_ref[...] = m[:, :_H]
    ev = ev_ref[...]
    for d, ref in enumerate((uv0_ref, uv1_ref, uv2_ref)):
        ref[...] = vecj[:, d * _H:(d + 1) * _H] * m2 + m3 * ev[:, d:d + 1]


def _msg(rows, rbfh, ev):
    be = 1600
    out_sds = jax.ShapeDtypeStruct((_E, _H), jnp.float32)
    return pl.pallas_call(
        _msg_body,
        grid=(_E // be,),
        in_specs=[
            pl.BlockSpec((be, _H3), lambda ii: (ii, 0)),
            pl.BlockSpec((be, _H3), lambda ii: (ii, 0)),
            pl.BlockSpec((be, 3), lambda ii: (ii, 0)),
        ],
        out_specs=[pl.BlockSpec((be, _H), lambda ii: (ii, 0))] * 4,
        out_shape=[out_sds] * 4,
    )(rows, rbfh, ev)


@functools.cache
def _sc_scatter_fn():
    mesh = plsc.VectorSubcoreMesh(core_axis_name="c", subcore_axis_name="s",
                                  num_cores=_NC, num_subcores=_NS)

    @functools.partial(
        pl.kernel,
        out_type=jax.ShapeDtypeStruct((4, _NC, _N, _H), jnp.float32),
        mesh=mesh,
        scratch_types=[
            pltpu.VMEM((_CHUNKS, _C), jnp.int32),
            pltpu.VMEM((_C, _H), jnp.float32),
            pltpu.VMEM((_C, _H), jnp.float32),
            pltpu.VMEM_SHARED((_N, _H), jnp.float32),
            pltpu.SemaphoreType.DMA,
            pltpu.SemaphoreType.DMA,
        ],
    )
    def _sc_scatter(u1, uv0, uv1, uv2, iidx3, zeros, out,
                    iall, ub0, ub1, acc, r0, r1):
        c = lax.axis_index("c")
        s = lax.axis_index("s")
        w = s * _NC + c
        rbase = s * _RPT

        tail = _NS * _RPT
        pltpu.sync_copy(iidx3.at[w], iall)

        def base(k):
            return w * _PERW + k * _C

        for p, u in enumerate((u1, uv0, uv1, uv2)):
            pltpu.sync_copy(zeros.at[pl.ds(rbase, _RPT)],
                            acc.at[pl.ds(rbase, _RPT)])

            @pl.when(s == _NS - 1)
            def _zero_tail():
                pltpu.sync_copy(zeros.at[pl.ds(tail, _RTAIL)],
                                acc.at[pl.ds(tail, _RTAIL)])

            plsc.subcore_barrier()

            def read(k, ub, rs, u=u):
                pltpu.async_copy(u.at[pl.ds(base(k), _C)], ub, rs)

            def wait_read(k, ub, rs, u=u):
                pltpu.make_async_copy(u.at[pl.ds(base(k), _C)], ub, rs).wait()

            def scat(k, ub):
                pltpu.sync_copy(ub, acc.at[iall.at[k]], add=True)

            read(0, ub0, r0)
            read(1, ub1, r1)

            def body(k2, carry):
                ka = 2 * k2
                kb = ka + 1
                wait_read(ka, ub0, r0)
                scat(ka, ub0)
                read(ka + 2, ub0, r0)
                wait_read(kb, ub1, r1)
                scat(kb, ub1)
                read(kb + 2, ub1, r1)
                return carry

            lax.fori_loop(0, _M2 - 1, body, 0)
            for k, ub, rs in ((_CHUNKS - 2, ub0, r0), (_CHUNKS - 1, ub1, r1)):
                wait_read(k, ub, rs)
                scat(k, ub)
            plsc.subcore_barrier()
            pltpu.sync_copy(acc.at[pl.ds(rbase, _RPT)],
                            out.at[p, c].at[pl.ds(rbase, _RPT)])

            @pl.when(s == _NS - 1)
            def _dump_tail():
                pltpu.sync_copy(acc.at[pl.ds(tail, _RTAIL)],
                                out.at[p, c].at[pl.ds(tail, _RTAIL)])

            plsc.subcore_barrier()

    return _sc_scatter


def kernel(x, vec, edge_index, edge_rbf, edge_vector, W1, b1, W2, b2, Wr, br,
           ln_g, ln_b):
    tab = _dense(x, vec.reshape(_N, _H3), W1, b1.reshape(1, -1), W2,
                 b2.reshape(1, -1), ln_g.reshape(1, -1), ln_b.reshape(1, -1))
    rbfh = _rbfh(edge_rbf, Wr, br.reshape(1, -1))
    j = edge_index[0].reshape(_NW, _CHUNKS, _C)
    i = edge_index[1].reshape(_NW, _CHUNKS, _C)
    rows = _sc_gather_fn()(tab, j)
    u1, uv0, uv1, uv2 = _msg(rows, rbfh, edge_vector)
    zeros = jnp.zeros((_N, _H), jnp.float32)
    parts = _sc_scatter_fn()(u1, uv0, uv1, uv2, i, zeros)
    dx = parts[0, 0] + parts[0, 1]
    dvec = jnp.stack([parts[1, 0] + parts[1, 1],
                      parts[2, 0] + parts[2, 1],
                      parts[3, 0] + parts[3, 1]], axis=1)
    return dx, dvec


# speedup vs baseline: 22.8828x; 1.1608x over previous
import functools

import jax
import jax.numpy as jnp
import numpy as np
from jax import lax
from jax.experimental import pallas as pl
from jax.experimental.pallas import tpu as pltpu
from jax.experimental.pallas import tpu_sc as plsc

_N = 10000
_E = 320000
_H = 128
_R = 32
_H3 = 3 * _H

_SCALED_SILU = 1.0 / 0.6
_INV_SQRT_3 = 1.0 / np.sqrt(3.0)
_INV_SQRT_H = 1.0 / np.sqrt(float(_H))

_NC = 2
_NS = 16
_NW = _NC * _NS
_PERW = _E // _NW
_C = 40
_CHUNKS = _PERW // _C
_M2 = _CHUNKS // 2
_RPT = 624
_RTAIL = _N - _NS * _RPT


def _dense_body(x_ref, vec_ref, w1_ref, b1_ref, w2_ref, b2_ref, g_ref, bb_ref,
                out_ref):
    x = x_ref[...]
    mu = jnp.mean(x, axis=-1, keepdims=True)
    var = jnp.mean((x - mu) ** 2, axis=-1, keepdims=True)
    xln = (x - mu) * lax.rsqrt(var + 1e-5) * g_ref[...] + bb_ref[...]
    h = lax.dot_general(xln, w1_ref[...], (((1,), (1,)), ((), ())),
                        preferred_element_type=jnp.float32) + b1_ref[...]
    h = h * jax.nn.sigmoid(h) * _SCALED_SILU
    xh = lax.dot_general(h, w2_ref[...], (((1,), (1,)), ((), ())),
                         preferred_element_type=jnp.float32) + b2_ref[...]
    scale = jnp.concatenate([
        jnp.ones((1, _H), jnp.float32),
        jnp.full((1, _H), _INV_SQRT_3 * _INV_SQRT_H, jnp.float32),
        jnp.full((1, _H), _INV_SQRT_H, jnp.float32),
    ], axis=1)
    lo = lax.bitcast_convert_type(
        (xh * scale).astype(jnp.bfloat16), jnp.uint16).astype(jnp.uint32)
    hi = lax.bitcast_convert_type(
        vec_ref[...].astype(jnp.bfloat16), jnp.uint16).astype(jnp.uint32)
    out_ref[...] = lax.bitcast_convert_type(lo | (hi << 16), jnp.int32)


def _dense(x, vecf, w1, b1, w2, b2, g, bb):
    bn = 2000
    return pl.pallas_call(
        _dense_body,
        grid=(_N // bn,),
        in_specs=[
            pl.BlockSpec((bn, _H), lambda ii: (ii, 0)),
            pl.BlockSpec((bn, _H3), lambda ii: (ii, 0)),
            pl.BlockSpec((_H, _H), lambda ii: (0, 0)),
            pl.BlockSpec((1, _H), lambda ii: (0, 0)),
            pl.BlockSpec((_H3, _H), lambda ii: (0, 0)),
            pl.BlockSpec((1, _H3), lambda ii: (0, 0)),
            pl.BlockSpec((1, _H), lambda ii: (0, 0)),
            pl.BlockSpec((1, _H), lambda ii: (0, 0)),
        ],
        out_specs=pl.BlockSpec((bn, _H3), lambda ii: (ii, 0)),
        out_shape=jax.ShapeDtypeStruct((_N, _H3), jnp.int32),
    )(x, vecf, w1, b1, w2, b2, g, bb)


@functools.cache
def _sc_gather_fn():
    mesh = plsc.VectorSubcoreMesh(core_axis_name="c", subcore_axis_name="s",
                                  num_cores=_NC, num_subcores=_NS)

    @functools.partial(
        pl.kernel,
        out_type=jax.ShapeDtypeStruct((_E, _H3), jnp.int32),
        mesh=mesh,
        scratch_types=[
            pltpu.VMEM((_CHUNKS, _C), jnp.int32),
            pltpu.VMEM((_C, _H3), jnp.int32),
            pltpu.VMEM((_C, _H3), jnp.int32),
            pltpu.SemaphoreType.DMA,
            pltpu.SemaphoreType.DMA,
            pltpu.SemaphoreType.DMA,
            pltpu.SemaphoreType.DMA,
        ],
    )
    def _sc_gather(tab, jidx3, rows_out, jall, xb0, xb1, g0, g1, w0, w1):
        c = lax.axis_index("c")
        s = lax.axis_index("s")
        w = s * _NC + c
        pltpu.sync_copy(jidx3.at[w], jall)

        def base(k):
            return w * _PERW + k * _C

        def start(k, xb, gs):
            pltpu.async_copy(tab.at[jall.at[k]], xb, gs)

        def wait_gather(k, xb, gs):
            pltpu.make_async_copy(tab.at[jall.at[k]], xb, gs).wait()

        def fire_write(k, xb, ws):
            pltpu.async_copy(xb, rows_out.at[pl.ds(base(k), _C)], ws)

        def wait_write(k, xb, ws):
            pltpu.make_async_copy(xb, rows_out.at[pl.ds(base(k), _C)], ws).wait()

        start(0, xb0, g0)
        start(1, xb1, g1)

        def body(k2, carry):
            ka = 2 * k2
            kb = ka + 1
            wait_gather(ka, xb0, g0)
            fire_write(ka, xb0, w0)
            wait_write(ka, xb0, w0)
            start(ka + 2, xb0, g0)
            wait_gather(kb, xb1, g1)
            fire_write(kb, xb1, w1)
            wait_write(kb, xb1, w1)
            start(kb + 2, xb1, g1)
            return carry

        lax.fori_loop(0, _M2 - 1, body, 0)
        for k, xb, gs, ws in ((_CHUNKS - 2, xb0, g0, w0),
                              (_CHUNKS - 1, xb1, g1, w1)):
            wait_gather(k, xb, gs)
            fire_write(k, xb, ws)
            wait_write(k, xb, ws)

    return _sc_gather


def _msg_body(rows_ref, rbf_ref, wr_ref, br_ref, ev_ref,
              u1_ref, uv0_ref, uv1_ref, uv2_ref):
    rbfh = lax.dot_general(
        rbf_ref[...], wr_ref[...], (((1,), (1,)), ((), ())),
        preferred_element_type=jnp.float32) + br_ref[...]
    ru = lax.bitcast_convert_type(rows_ref[...], jnp.uint32)
    lo16 = lax.convert_element_type(ru & 0xFFFF, jnp.uint16)
    hi16 = lax.convert_element_type(ru >> 16, jnp.uint16)
    xhj = lax.bitcast_convert_type(lo16, jnp.bfloat16).astype(jnp.float32)
    vecj = lax.bitcast_convert_type(hi16, jnp.bfloat16).astype(jnp.float32)
    m = xhj * rbfh
    m2 = m[:, _H:2 * _H]
    m3 = m[:, 2 * _H:]
    u1_ref[...] = m[:, :_H]
    ev = ev_ref[...]
    for d, ref in enumerate((uv0_ref, uv1_ref, uv2_ref)):
        ref[...] = vecj[:, d * _H:(d + 1) * _H] * m2 + m3 * ev[:, d:d + 1]


def _msg(rows, rbf, wr, br, ev):
    be = 1600
    out_sds = jax.ShapeDtypeStruct((_E, _H), jnp.float32)
    return pl.pallas_call(
        _msg_body,
        grid=(_E // be,),
        in_specs=[
            pl.BlockSpec((be, _H3), lambda ii: (ii, 0)),
            pl.BlockSpec((be, _R), lambda ii: (ii, 0)),
            pl.BlockSpec((_H3, _R), lambda ii: (0, 0)),
            pl.BlockSpec((1, _H3), lambda ii: (0, 0)),
            pl.BlockSpec((be, 3), lambda ii: (ii, 0)),
        ],
        out_specs=[pl.BlockSpec((be, _H), lambda ii: (ii, 0))] * 4,
        out_shape=[out_sds] * 4,
    )(rows, rbf, wr, br, ev)


@functools.cache
def _sc_scatter_fn():
    mesh = plsc.VectorSubcoreMesh(core_axis_name="c", subcore_axis_name="s",
                                  num_cores=_NC, num_subcores=_NS)

    @functools.partial(
        pl.kernel,
        out_type=jax.ShapeDtypeStruct((4, _NC, _N, _H), jnp.float32),
        mesh=mesh,
        scratch_types=[
            pltpu.VMEM((_CHUNKS, _C), jnp.int32),
            pltpu.VMEM((_C, _H), jnp.float32),
            pltpu.VMEM((_C, _H), jnp.float32),
            pltpu.VMEM_SHARED((_N, _H), jnp.float32),
            pltpu.SemaphoreType.DMA,
            pltpu.SemaphoreType.DMA,
        ],
    )
    def _sc_scatter(u1, uv0, uv1, uv2, iidx3, zeros, out,
                    iall, ub0, ub1, acc, r0, r1):
        c = lax.axis_index("c")
        s = lax.axis_index("s")
        w = s * _NC + c
        rbase = s * _RPT

        tail = _NS * _RPT
        pltpu.sync_copy(iidx3.at[w], iall)

        def base(k):
            return w * _PERW + k * _C

        for p, u in enumerate((u1, uv0, uv1, uv2)):
            pltpu.sync_copy(zeros.at[pl.ds(rbase, _RPT)],
                            acc.at[pl.ds(rbase, _RPT)])

            @pl.when(s == _NS - 1)
            def _zero_tail():
                pltpu.sync_copy(zeros.at[pl.ds(tail, _RTAIL)],
                                acc.at[pl.ds(tail, _RTAIL)])

            plsc.subcore_barrier()

            def read(k, ub, rs, u=u):
                pltpu.async_copy(u.at[pl.ds(base(k), _C)], ub, rs)

            def wait_read(k, ub, rs, u=u):
                pltpu.make_async_copy(u.at[pl.ds(base(k), _C)], ub, rs).wait()

            def scat(k, ub):
                pltpu.sync_copy(ub, acc.at[iall.at[k]], add=True)

            read(0, ub0, r0)
            read(1, ub1, r1)

            def body(k2, carry):
                ka = 2 * k2
                kb = ka + 1
                wait_read(ka, ub0, r0)
                scat(ka, ub0)
                read(ka + 2, ub0, r0)
                wait_read(kb, ub1, r1)
                scat(kb, ub1)
                read(kb + 2, ub1, r1)
                return carry

            lax.fori_loop(0, _M2 - 1, body, 0)
            for k, ub, rs in ((_CHUNKS - 2, ub0, r0), (_CHUNKS - 1, ub1, r1)):
                wait_read(k, ub, rs)
                scat(k, ub)
            plsc.subcore_barrier()
            pltpu.sync_copy(acc.at[pl.ds(rbase, _RPT)],
                            out.at[p, c].at[pl.ds(rbase, _RPT)])

            @pl.when(s == _NS - 1)
            def _dump_tail():
                pltpu.sync_copy(acc.at[pl.ds(tail, _RTAIL)],
                                out.at[p, c].at[pl.ds(tail, _RTAIL)])

            plsc.subcore_barrier()

    return _sc_scatter


def kernel(x, vec, edge_index, edge_rbf, edge_vector, W1, b1, W2, b2, Wr, br,
           ln_g, ln_b):
    tab = _dense(x, vec.reshape(_N, _H3), W1, b1.reshape(1, -1), W2,
                 b2.reshape(1, -1), ln_g.reshape(1, -1), ln_b.reshape(1, -1))
    j = edge_index[0].reshape(_NW, _CHUNKS, _C)
    i = edge_index[1].reshape(_NW, _CHUNKS, _C)
    rows = _sc_gather_fn()(tab, j)
    u1, uv0, uv1, uv2 = _msg(rows, edge_rbf, Wr, br.reshape(1, -1), edge_vector)
    zeros = jnp.zeros((_N, _H), jnp.float32)
    parts = _sc_scatter_fn()(u1, uv0, uv1, uv2, i, zeros)
    dx = parts[0, 0] + parts[0, 1]
    dvec = jnp.stack([parts[1, 0] + parts[1, 1],
                      parts[2, 0] + parts[2, 1],
                      parts[3, 0] + parts[3, 1]], axis=1)
    return dx, dvec


# speedup vs baseline: 26.8266x; 1.1723x over previous
import functools

import jax
import jax.numpy as jnp
import numpy as np
from jax import lax
from jax.experimental import pallas as pl
from jax.experimental.pallas import tpu as pltpu
from jax.experimental.pallas import tpu_sc as plsc

_N = 10000
_E = 320000
_H = 128
_R = 32
_H3 = 3 * _H

_SCALED_SILU = 1.0 / 0.6
_INV_SQRT_3 = 1.0 / np.sqrt(3.0)
_INV_SQRT_H = 1.0 / np.sqrt(float(_H))

_NC = 2
_NS = 16
_NW = _NC * _NS
_PERW = _E // _NW
_C = 40
_CHUNKS = _PERW // _C
_CS = 80
_SCHUNKS = _PERW // _CS
_RPT = 624
_RTAIL = _N - _NS * _RPT


def _dense_body(x_ref, vec_ref, w1_ref, b1_ref, w2_ref, b2_ref, g_ref, bb_ref,
                out_ref):
    x = x_ref[...]
    mu = jnp.mean(x, axis=-1, keepdims=True)
    var = jnp.mean((x - mu) ** 2, axis=-1, keepdims=True)
    xln = (x - mu) * lax.rsqrt(var + 1e-5) * g_ref[...] + bb_ref[...]
    h = lax.dot_general(xln, w1_ref[...], (((1,), (1,)), ((), ())),
                        preferred_element_type=jnp.float32) + b1_ref[...]
    h = h * jax.nn.sigmoid(h) * _SCALED_SILU
    xh = lax.dot_general(h, w2_ref[...], (((1,), (1,)), ((), ())),
                         preferred_element_type=jnp.float32) + b2_ref[...]
    scale = jnp.concatenate([
        jnp.ones((1, _H), jnp.float32),
        jnp.full((1, _H), _INV_SQRT_3 * _INV_SQRT_H, jnp.float32),
        jnp.full((1, _H), _INV_SQRT_H, jnp.float32),
    ], axis=1)
    lo = lax.bitcast_convert_type(
        (xh * scale).astype(jnp.bfloat16), jnp.uint16).astype(jnp.uint32)
    hi = lax.bitcast_convert_type(
        vec_ref[...].astype(jnp.bfloat16), jnp.uint16).astype(jnp.uint32)
    out_ref[...] = lax.bitcast_convert_type(lo | (hi << 16), jnp.int32)


def _dense(x, vecf, w1, b1, w2, b2, g, bb):
    bn = 2000
    return pl.pallas_call(
        _dense_body,
        grid=(_N // bn,),
        in_specs=[
            pl.BlockSpec((bn, _H), lambda ii: (ii, 0)),
            pl.BlockSpec((bn, _H3), lambda ii: (ii, 0)),
            pl.BlockSpec((_H, _H), lambda ii: (0, 0)),
            pl.BlockSpec((1, _H), lambda ii: (0, 0)),
            pl.BlockSpec((_H3, _H), lambda ii: (0, 0)),
            pl.BlockSpec((1, _H3), lambda ii: (0, 0)),
            pl.BlockSpec((1, _H), lambda ii: (0, 0)),
            pl.BlockSpec((1, _H), lambda ii: (0, 0)),
        ],
        out_specs=pl.BlockSpec((bn, _H3), lambda ii: (ii, 0)),
        out_shape=jax.ShapeDtypeStruct((_N, _H3), jnp.int32),
    )(x, vecf, w1, b1, w2, b2, g, bb)


@functools.cache
def _sc_gather_fn():
    mesh = plsc.VectorSubcoreMesh(core_axis_name="c", subcore_axis_name="s",
                                  num_cores=_NC, num_subcores=_NS)

    @functools.partial(
        pl.kernel,
        out_type=jax.ShapeDtypeStruct((_E, _H3), jnp.int32),
        mesh=mesh,
        scratch_types=[
            pltpu.VMEM((_CHUNKS, _C), jnp.int32),
            pltpu.VMEM((4, _C, _H3), jnp.int32),
            pltpu.SemaphoreType.DMA,
            pltpu.SemaphoreType.DMA,
            pltpu.SemaphoreType.DMA,
            pltpu.SemaphoreType.DMA,
            pltpu.SemaphoreType.DMA,
            pltpu.SemaphoreType.DMA,
            pltpu.SemaphoreType.DMA,
            pltpu.SemaphoreType.DMA,
        ],
    )
    def _sc_gather(tab, jidx3, rows_out, jall, bufs,
                   g0, g1, g2, g3, w0, w1, w2, w3):
        c = lax.axis_index("c")
        s = lax.axis_index("s")
        w = s * _NC + c
        pltpu.sync_copy(jidx3.at[w], jall)
        gsem = (g0, g1, g2, g3)
        wsem = (w0, w1, w2, w3)

        def base(k):
            return w * _PERW + k * _C

        def start(k, b):
            pltpu.async_copy(tab.at[jall.at[k]], bufs.at[b], gsem[b])

        def wait_gather(k, b):
            pltpu.make_async_copy(tab.at[jall.at[k]], bufs.at[b],
                                  gsem[b]).wait()

        def fire_write(k, b):
            pltpu.async_copy(bufs.at[b], rows_out.at[pl.ds(base(k), _C)],
                             wsem[b])

        def wait_write(k, b):
            pltpu.make_async_copy(bufs.at[b], rows_out.at[pl.ds(base(k), _C)],
                                  wsem[b]).wait()

        start(0, 0)
        start(1, 1)
        for t in (0, 1):
            wait_gather(t, t % 4)
            fire_write(t, t % 4)
            start(t + 2, (t + 2) % 4)

        def body(q, carry):
            for bb in range(4):
                t = 2 + 4 * q + bb
                sl = (2 + bb) % 4
                wait_gather(t, sl)
                fire_write(t, sl)
                wait_write(t - 2, sl_prev := (sl + 2) % 4)
                start(t + 2, sl_prev)
            return carry

        lax.fori_loop(0, (_CHUNKS - 6) // 4, body, 0)
        for t in range(_CHUNKS - 4, _CHUNKS):
            sl = t % 4
            wait_gather(t, sl)
            fire_write(t, sl)
            if t + 2 < _CHUNKS:
                wait_write(t - 2, (sl + 2) % 4)
                start(t + 2, (sl + 2) % 4)
        for t in range(_CHUNKS - 4, _CHUNKS):
            wait_write(t, t % 4)

    return _sc_gather


def _msg_body(rows_ref, rbf_ref, wr_ref, br_ref, ev_ref,
              u1_ref, uv0_ref, uv1_ref, uv2_ref):
    rbfh = lax.dot_general(
        rbf_ref[...], wr_ref[...], (((1,), (1,)), ((), ())),
        preferred_element_type=jnp.float32) + br_ref[...]
    ru = lax.bitcast_convert_type(rows_ref[...], jnp.uint32)
    lo16 = lax.convert_element_type(ru & 0xFFFF, jnp.uint16)
    hi16 = lax.convert_element_type(ru >> 16, jnp.uint16)
    xhj = lax.bitcast_convert_type(lo16, jnp.bfloat16).astype(jnp.float32)
    vecj = lax.bitcast_convert_type(hi16, jnp.bfloat16).astype(jnp.float32)
    m = xhj * rbfh
    m2 = m[:, _H:2 * _H]
    m3 = m[:, 2 * _H:]
    u1_ref[...] = m[:, :_H]
    ev = ev_ref[...]
    for d, ref in enumerate((uv0_ref, uv1_ref, uv2_ref)):
        ref[...] = vecj[:, d * _H:(d + 1) * _H] * m2 + m3 * ev[:, d:d + 1]


def _msg(rows, rbf, wr, br, ev):
    be = 1600
    out_sds = jax.ShapeDtypeStruct((_E, _H), jnp.float32)
    return pl.pallas_call(
        _msg_body,
        grid=(_E // be,),
        in_specs=[
            pl.BlockSpec((be, _H3), lambda ii: (ii, 0)),
            pl.BlockSpec((be, _R), lambda ii: (ii, 0)),
            pl.BlockSpec((_H3, _R), lambda ii: (0, 0)),
            pl.BlockSpec((1, _H3), lambda ii: (0, 0)),
            pl.BlockSpec((be, 3), lambda ii: (ii, 0)),
        ],
        out_specs=[pl.BlockSpec((be, _H), lambda ii: (ii, 0))] * 4,
        out_shape=[out_sds] * 4,
    )(rows, rbf, wr, br, ev)


@functools.cache
def _sc_scatter_fn():
    mesh = plsc.VectorSubcoreMesh(core_axis_name="c", subcore_axis_name="s",
                                  num_cores=_NC, num_subcores=_NS)

    @functools.partial(
        pl.kernel,
        out_type=jax.ShapeDtypeStruct((4, _NC, _N, _H), jnp.float32),
        mesh=mesh,
        scratch_types=[
            pltpu.VMEM((_SCHUNKS, _CS), jnp.int32),
            pltpu.VMEM((3, _CS, _H), jnp.float32),
            pltpu.VMEM_SHARED((_N, _H), jnp.float32),
            pltpu.SemaphoreType.DMA,
            pltpu.SemaphoreType.DMA,
            pltpu.SemaphoreType.DMA,
        ],
    )
    def _sc_scatter(u1, uv0, uv1, uv2, iidx3, zeros, out,
                    iall, bufs, acc, r0, r1, r2):
        c = lax.axis_index("c")
        s = lax.axis_index("s")
        w = s * _NC + c
        rbase = s * _RPT
        rsem = (r0, r1, r2)

        tail = _NS * _RPT
        pltpu.sync_copy(iidx3.at[w], iall)

        def base(k):
            return w * _PERW + k * _CS

        for p, u in enumerate((u1, uv0, uv1, uv2)):
            pltpu.sync_copy(zeros.at[pl.ds(rbase, _RPT)],
                            acc.at[pl.ds(rbase, _RPT)])

            @pl.when(s == _NS - 1)
            def _zero_tail():
                pltpu.sync_copy(zeros.at[pl.ds(tail, _RTAIL)],
                                acc.at[pl.ds(tail, _RTAIL)])

            plsc.subcore_barrier()

            def read(k, b, u=u):
                pltpu.async_copy(u.at[pl.ds(base(k), _CS)], bufs.at[b],
                                 rsem[b])

            def wait_read(k, b, u=u):
                pltpu.make_async_copy(u.at[pl.ds(base(k), _CS)], bufs.at[b],
                                      rsem[b]).wait()

            def scat(k, b):
                pltpu.sync_copy(bufs.at[b], acc.at[iall.at[k]], add=True)

            for t in range(3):
                read(t, t)

            def body(q, carry):
                for bb in range(3):
                    t = 3 * q + bb
                    wait_read(t, bb)
                    scat(t, bb)
                    read(t + 3, bb)
                return carry

            lax.fori_loop(0, (_SCHUNKS - 4) // 3, body, 0)
            for t in range(3 * ((_SCHUNKS - 4) // 3), _SCHUNKS):
                sl = t % 3
                wait_read(t, sl)
                scat(t, sl)
                if t + 3 < _SCHUNKS:
                    read(t + 3, sl)
            plsc.subcore_barrier()
            pltpu.sync_copy(acc.at[pl.ds(rbase, _RPT)],
                            out.at[p, c].at[pl.ds(rbase, _RPT)])

            @pl.when(s == _NS - 1)
            def _dump_tail():
                pltpu.sync_copy(acc.at[pl.ds(tail, _RTAIL)],
                                out.at[p, c].at[pl.ds(tail, _RTAIL)])

            plsc.subcore_barrier()

    return _sc_scatter


def kernel(x, vec, edge_index, edge_rbf, edge_vector, W1, b1, W2, b2, Wr, br,
           ln_g, ln_b):
    tab = _dense(x, vec.reshape(_N, _H3), W1, b1.reshape(1, -1), W2,
                 b2.reshape(1, -1), ln_g.reshape(1, -1), ln_b.reshape(1, -1))
    j = edge_index[0].reshape(_NW, _CHUNKS, _C)
    i = edge_index[1].reshape(_NW, _SCHUNKS, _CS)
    rows = _sc_gather_fn()(tab, j)
    u1, uv0, uv1, uv2 = _msg(rows, edge_rbf, Wr, br.reshape(1, -1), edge_vector)
    zeros = jnp.zeros((_N, _H), jnp.float32)
    parts = _sc_scatter_fn()(u1, uv0, uv1, uv2, i, zeros)
    dx = parts[0, 0] + parts[0, 1]
    dvec = jnp.stack([parts[1, 0] + parts[1, 1],
                      parts[2, 0] + parts[2, 1],
                      parts[3, 0] + parts[3, 1]], axis=1)
    return dx, dvec
